# baseline (speedup 1.0000x reference)
import functools

import jax
import jax.numpy as jnp
from jax import lax
from jax.experimental import pallas as pl
from jax.experimental.pallas import tpu as pltpu
from jax.experimental.pallas import tpu_sc as plsc

SELF_LOOP_ID = 38
N_NODES = 10000
N_EDGES = 320000
D = 128
N_REL = 38
N_SENT = 16
IDS_PER_SENT = 32

NC, NS = 2, 16
NW = NC * NS
LN = 16

NODES_PAD = 10240
NPT = NODES_PAD // NW
EPT = N_EDGES // NW
EPT_PAD = 10240
CH1 = 128
NPAIR = EPT_PAD // CH1 // 2
CH = 80
NODE_HALF = NODES_PAD // 2
AGG_ROWS = NODE_HALF + 8
ROWS_PT = NODE_HALF // NS
WB = 80

N_TRI = N_EDGES + N_NODES
TRI_PAD = 330240
TPT = TRI_PAD // NW
NCH2 = TPT // CH
NCH2_LAST = (N_TRI - (NW - 1) * TPT) // CH

_MESH = plsc.VectorSubcoreMesh(
    core_axis_name="c", subcore_axis_name="s", num_cores=NC, num_subcores=NS)
_SC_PARAMS = pltpu.CompilerParams(needs_layout_passes=False)


def _msgpass_body(ce_hbm, cid_hbm, src_hbm, dst_hbm, rel_hbm, w_hbm,
                  relemb_hbm,
                  x0_hbm, agg_hbm,
                  cidv, src1, dst1, rel1, w1, relv, bufA, bufB,
                  gsrcA, gsrcB, dstA, dstB,
                  agg_sp, semA, semB, semSA, semSB):
    c = lax.axis_index("c")
    s = lax.axis_index("s")
    wid = c * NS + s

    pltpu.sync_copy(cid_hbm, cidv)
    eb = wid * EPT_PAD
    pltpu.sync_copy(src_hbm.at[pl.ds(eb, EPT_PAD)], src1)
    pltpu.sync_copy(dst_hbm.at[pl.ds(eb, EPT_PAD)], dst1)
    pltpu.sync_copy(rel_hbm.at[pl.ds(eb, EPT_PAD)], rel1)
    pltpu.sync_copy(w_hbm.at[pl.ds(eb, EPT_PAD)], w1)
    pltpu.sync_copy(relemb_hbm, relv)

    for q in range(NPT // 80):
        base = wid * NPT + q * 80
        pltpu.async_copy(ce_hbm.at[cidv.at[pl.ds(base, 80)]],
                         bufA.at[pl.ds(0, 80)], semA).wait()
        pltpu.sync_copy(bufA.at[pl.ds(0, 80)], x0_hbm.at[pl.ds(base, 80)])

    zv = jnp.zeros((LN,), jnp.float32)

    def zrow(r, cc):
        for j in range(D // LN):
            bufA[r, pl.ds(j * LN, LN)] = zv
        return cc

    def _stage(kb, gsrcv, dstcv, p):
        for i in range(CH1 // LN):
            sv = src1[pl.ds(kb + i * LN, LN)]
            gsrcv[pl.ds(i * LN, LN)] = plsc.load_gather(cidv, [sv])
            dd = dst1[pl.ds(kb + i * LN, LN)] - (p * NODE_HALF)
            ok = (dd >= 0) & (dd < NODE_HALF)
            dstcv[pl.ds(i * LN, LN)] = jnp.where(
                ok, dd, jnp.full((LN,), NODE_HALF, jnp.int32))

    def _compute(kb, buf):
        def grp(g, cc):
            gb = kb + g * LN
            wvec = w1[pl.ds(gb, LN)]
            rvec = rel1[pl.ds(gb, LN)] * D
            for e16 in range(LN):
                e = g * LN + e16
                w_e = wvec[e16]
                rb = rvec[e16]
                for j in range(D // LN):
                    xv = buf[e, pl.ds(j * LN, LN)]
                    rv = relv[pl.ds(rb + j * LN, LN)]
                    buf[e, pl.ds(j * LN, LN)] = jnp.maximum(
                        xv + w_e * rv, 0.0)
            return cc

        lax.fori_loop(0, CH1 // LN, grp, 0)

    for p in range(2):
        lax.fori_loop(0, WB, zrow, 0)
        for q in range(ROWS_PT // WB):
            pltpu.sync_copy(bufA.at[pl.ds(0, WB)],
                            agg_sp.at[pl.ds(s * ROWS_PT + q * WB, WB)])
        plsc.subcore_barrier()

        def pair(m, carry):
            ka = m * 2 * CH1
            kb = ka + CH1
            _stage(ka, gsrcA, dstA, p)
            cpA = pltpu.async_copy(ce_hbm.at[gsrcA], bufA, semA)
            _stage(kb, gsrcB, dstB, p)
            cpB = pltpu.async_copy(ce_hbm.at[gsrcB], bufB, semB)
            cpA.wait()
            _compute(ka, bufA)
            scA = pltpu.async_copy(bufA, agg_sp.at[dstA], semSA, add=True)
            cpB.wait()
            _compute(kb, bufB)
            scB = pltpu.async_copy(bufB, agg_sp.at[dstB], semSB, add=True)
            scA.wait()
            scB.wait()
            return carry

        lax.fori_loop(0, NPAIR, pair, 0)
        plsc.subcore_barrier()
        for q in range(ROWS_PT // WB):
            rb = s * ROWS_PT + q * WB
            pltpu.sync_copy(agg_sp.at[pl.ds(rb, WB)], bufA.at[pl.ds(0, WB)])
            pltpu.sync_copy(
                bufA.at[pl.ds(0, WB)],
                agg_hbm.at[pl.ds((c * 2 + p) * NODE_HALF + rb, WB)])
        if p == 0:
            plsc.subcore_barrier()


_msgpass = functools.partial(
    pl.kernel,
    out_type=[
        pltpu.HBM((NODES_PAD, D), jnp.float32),
        pltpu.HBM((NC * 2 * NODE_HALF, D), jnp.float32),
    ],
    mesh=_MESH,
    scratch_types=[
        pltpu.VMEM((NODES_PAD,), jnp.int32),
        pltpu.VMEM((EPT_PAD,), jnp.int32),
        pltpu.VMEM((EPT_PAD,), jnp.int32),
        pltpu.VMEM((EPT_PAD,), jnp.int32),
        pltpu.VMEM((EPT_PAD,), jnp.float32),
        pltpu.VMEM((N_REL * D,), jnp.float32),
        pltpu.VMEM((CH1, D), jnp.float32),
        pltpu.VMEM((CH1, D), jnp.float32),
        pltpu.VMEM((CH1,), jnp.int32),
        pltpu.VMEM((CH1,), jnp.int32),
        pltpu.VMEM((CH1,), jnp.int32),
        pltpu.VMEM((CH1,), jnp.int32),
        pltpu.VMEM_SHARED((AGG_ROWS, D), jnp.float32),
        pltpu.SemaphoreType.DMA,
        pltpu.SemaphoreType.DMA,
        pltpu.SemaphoreType.DMA,
        pltpu.SemaphoreType.DMA,
    ],
    compiler_params=_SC_PARAMS,
)(_msgpass_body)


def _dense_body(x0_ref, agg_ref, W1_ref, b1_ref, W2_ref, b2_ref, Wl_ref,
                bl_ref, relsl_ref, cid2_ref, sent_ref,
                A_ref, C_ref, R2_ref, nm_ref):
    f32 = jnp.float32
    xin = (x0_ref[0:N_NODES] + agg_ref[0:N_NODES]
           + agg_ref[2 * NODE_HALF:2 * NODE_HALF + N_NODES])
    h = jnp.maximum(
        jnp.dot(xin, W1_ref[...], preferred_element_type=f32) + b1_ref[...], 0.0)
    x = jnp.dot(h, W2_ref[...], preferred_element_type=f32) + b2_ref[...]
    Wl = Wl_ref[...]
    A_ref[...] = jnp.dot(x, Wl[0:D], preferred_element_type=f32) + bl_ref[...]
    C_ref[...] = jnp.dot(x, Wl[2 * D:3 * D], preferred_element_type=f32)
    R2_ref[...] = jnp.dot(relsl_ref[...], Wl[D:2 * D], preferred_element_type=f32)
    cid2 = cid2_ref[...]
    nm = jnp.zeros_like(cid2)
    for si in range(N_SENT):
        acc = None
        for ii in range(IDS_PER_SENT):
            eq = cid2 == sent_ref[si, ii]
            acc = eq if acc is None else (acc | eq)
        nm = nm | (acc.astype(jnp.int32) << si)
    nm_ref[...] = nm


_dense = pl.pallas_call(
    _dense_body,
    out_shape=[
        jax.ShapeDtypeStruct((N_NODES, D), jnp.float32),
        jax.ShapeDtypeStruct((N_NODES, D), jnp.float32),
        jax.ShapeDtypeStruct((40, D), jnp.float32),
        jax.ShapeDtypeStruct((NODES_PAD // D, D), jnp.int32),
    ],
    in_specs=[
        pl.BlockSpec(memory_space=pltpu.VMEM),
        pl.BlockSpec(memory_space=pltpu.VMEM),
        pl.BlockSpec(memory_space=pltpu.VMEM),
        pl.BlockSpec(memory_space=pltpu.VMEM),
        pl.BlockSpec(memory_space=pltpu.VMEM),
        pl.BlockSpec(memory_space=pltpu.VMEM),
        pl.BlockSpec(memory_space=pltpu.VMEM),
        pl.BlockSpec(memory_space=pltpu.VMEM),
        pl.BlockSpec(memory_space=pltpu.VMEM),
        pl.BlockSpec(memory_space=pltpu.VMEM),
        pl.BlockSpec(memory_space=pltpu.SMEM),
    ],
)


def _asm_body(A_hbm, C_hbm, r2_hbm, cid_hbm, nm_hbm, h_hbm, t_hbm, r_hbm,
              w_hbm,
              enc_hbm, tid_hbm, mb_hbm,
              cidv, nmv, r2v, h1, t1, r1, w1, bA, bC, hc, tcv, mbb, tidf,
              semA, semB):
    c = lax.axis_index("c")
    s = lax.axis_index("s")
    wid = c * NS + s

    pltpu.sync_copy(cid_hbm, cidv)
    pltpu.sync_copy(nm_hbm, nmv)
    pltpu.sync_copy(r2_hbm, r2v)
    tb = wid * TPT
    pltpu.sync_copy(h_hbm.at[pl.ds(tb, TPT)], h1)
    pltpu.sync_copy(t_hbm.at[pl.ds(tb, TPT)], t1)
    pltpu.sync_copy(r_hbm.at[pl.ds(tb, TPT)], r1)
    pltpu.sync_copy(w_hbm.at[pl.ds(tb, TPT)], w1)

    iota3 = lax.iota(jnp.int32, LN) * 3

    def chunk(k, carry):
        kb = k * CH
        for i in range(CH // LN):
            hc[pl.ds(i * LN, LN)] = h1[pl.ds(kb + i * LN, LN)]
            tcv[pl.ds(i * LN, LN)] = t1[pl.ds(kb + i * LN, LN)]
        cpA = pltpu.async_copy(A_hbm.at[hc], bA, semA)
        cpB = pltpu.async_copy(C_hbm.at[tcv], bC, semB)
        for i in range(CH // LN):
            hv = hc[pl.ds(i * LN, LN)]
            tv = tcv[pl.ds(i * LN, LN)]
            chd = plsc.load_gather(cidv, [hv])
            ctl = plsc.load_gather(cidv, [tv])
            nh = plsc.load_gather(nmv, [hv])
            nt = plsc.load_gather(nmv, [tv])
            rv = r1[pl.ds(kb + i * LN, LN)]
            mbb[pl.ds(i * LN, LN)] = nh | nt
            base = iota3 + (3 * LN * i)
            plsc.store_scatter(tidf, [base], chd)
            plsc.store_scatter(tidf, [base + 1], rv)
            plsc.store_scatter(tidf, [base + 2], ctl)
        cpA.wait()
        cpB.wait()

        def grp(g, cc):
            gb = kb + g * LN
            wvec = w1[pl.ds(gb, LN)]
            rvec = r1[pl.ds(gb, LN)] * D
            for e16 in range(LN):
                e = g * LN + e16
                w_e = wvec[e16]
                rb = rvec[e16]
                for j in range(D // LN):
                    av = bA[e, pl.ds(j * LN, LN)]
                    cv = bC[e, pl.ds(j * LN, LN)]
                    rv2 = r2v[pl.ds(rb + j * LN, LN)]
                    bA[e, pl.ds(j * LN, LN)] = av + cv + w_e * rv2
            return cc

        lax.fori_loop(0, CH // LN, grp, 0)
        g0 = tb + kb
        pltpu.sync_copy(bA, enc_hbm.at[pl.ds(g0, CH)])
        pltpu.sync_copy(tidf, tid_hbm.at[pl.ds(g0 * 3, CH * 3)])
        pltpu.sync_copy(mbb, mb_hbm.at[pl.ds(g0, CH)])
        return carry

    nch = jnp.where(wid == NW - 1, NCH2_LAST, NCH2)
    lax.fori_loop(0, nch, chunk, 0)


_assemble = functools.partial(
    pl.kernel,
    out_type=[
        pltpu.HBM((N_TRI, D), jnp.float32),
        pltpu.HBM((N_TRI * 3,), jnp.int32),
        pltpu.HBM((N_TRI,), jnp.int32),
    ],
    mesh=_MESH,
    scratch_types=[
        pltpu.VMEM((NODES_PAD,), jnp.int32),
        pltpu.VMEM((NODES_PAD,), jnp.int32),
        pltpu.VMEM((40 * D,), jnp.float32),
        pltpu.VMEM((TPT,), jnp.int32),
        pltpu.VMEM((TPT,), jnp.int32),
        pltpu.VMEM((TPT,), jnp.int32),
        pltpu.VMEM((TPT,), jnp.float32),
        pltpu.VMEM((CH, D), jnp.float32),
        pltpu.VMEM((CH, D), jnp.float32),
        pltpu.VMEM((CH,), jnp.int32),
        pltpu.VMEM((CH,), jnp.int32),
        pltpu.VMEM((CH,), jnp.int32),
        pltpu.VMEM((CH * 3,), jnp.int32),
        pltpu.SemaphoreType.DMA,
        pltpu.SemaphoreType.DMA,
    ],
    compiler_params=_SC_PARAMS,
)(_asm_body)


def _mask_body(mb_ref, mask_ref):
    bits = mb_ref[...].reshape(1, N_TRI)
    shifts = jax.lax.broadcasted_iota(jnp.int32, (N_SENT, N_TRI), 0)
    mask_ref[...] = ((bits >> shifts) & 1) != 0


_mask_unpack = pl.pallas_call(
    _mask_body,
    out_shape=jax.ShapeDtypeStruct((N_SENT, N_TRI), jnp.bool_),
)


_EB = 13200


def _strip_body(enc_in, tid_in, enc_out, tid_out):
    enc_out[...] = enc_in[...]
    tid_out[...] = tid_in[...]


_strip = pl.pallas_call(
    _strip_body,
    grid=(N_TRI // _EB,),
    in_specs=[
        pl.BlockSpec((_EB, D), lambda i: (i, 0)),
        pl.BlockSpec((_EB, 3), lambda i: (i, 0)),
    ],
    out_specs=[
        pl.BlockSpec((_EB, D), lambda i: (i, 0)),
        pl.BlockSpec((_EB, 3), lambda i: (i, 0)),
    ],
    out_shape=[
        jax.ShapeDtypeStruct((N_TRI, D), jnp.float32),
        jax.ShapeDtypeStruct((N_TRI, 3), jnp.int32),
    ],
)


def kernel(concept_ids, edge_index, edge_relation, edge_weight,
           sent_concept_ids, concept_embedding, relation_embedding,
           self_loop_embedding, W1, b1, W2, b2, W_lin, b_lin):
    i32, f32 = jnp.int32, jnp.float32
    cid = concept_ids.astype(i32)
    src = edge_index[0].astype(i32)
    dst = edge_index[1].astype(i32)
    rel = edge_relation.astype(i32)
    w = edge_weight.astype(f32)

    cid_pad = jnp.concatenate([cid, jnp.zeros((NODES_PAD - N_NODES,), i32)])
    relv_flat = relation_embedding.astype(f32).reshape(-1)

    epad = NW * EPT_PAD - N_EDGES
    src_p = jnp.concatenate([src, jnp.zeros((epad,), i32)])
    dst_p = jnp.concatenate([dst, jnp.full((epad,), NODES_PAD, i32)])
    rel_p = jnp.concatenate([rel, jnp.zeros((epad,), i32)])
    w_p = jnp.concatenate([w, jnp.zeros((epad,), f32)])
    x0h, aggf = _msgpass(concept_embedding.astype(f32), cid_pad, src_p,
                         dst_p, rel_p, w_p, relv_flat)

    relsl = jnp.concatenate(
        [relation_embedding.astype(f32), self_loop_embedding.astype(f32),
         jnp.zeros((1, D), f32)], axis=0)
    cid2d = cid_pad.reshape(NODES_PAD // D, D)
    A, C, R2, nm2d = _dense(
        x0h, aggf, W1.astype(f32), b1.astype(f32).reshape(1, D),
        W2.astype(f32), b2.astype(f32).reshape(1, D), W_lin.astype(f32),
        b_lin.astype(f32).reshape(1, D), relsl, cid2d,
        sent_concept_ids.astype(i32))

    nm_flat = nm2d.reshape(-1)
    ar_n = jnp.arange(N_NODES, dtype=i32)
    npad = TRI_PAD - N_TRI
    h_ext = jnp.concatenate([src, ar_n, jnp.zeros((npad,), i32)])
    t_ext = jnp.concatenate([dst, ar_n, jnp.zeros((npad,), i32)])
    r_ext = jnp.concatenate(
        [rel, jnp.full((N_NODES,), SELF_LOOP_ID, i32), jnp.zeros((npad,), i32)])
    w_ext = jnp.concatenate(
        [w, jnp.ones((N_NODES,), f32), jnp.zeros((npad,), f32)])

    enc, tid_flat, mb = _assemble(
        A, C, R2.reshape(-1), cid_pad, nm_flat, h_ext, t_ext, r_ext, w_ext)

    mask = _mask_unpack(mb)
    enc, tid = _strip(enc, tid_flat.reshape(N_TRI, 3))
    return enc, mask, tid


# speedup vs baseline: 1.0023x; 1.0023x over previous
import functools

import jax
import jax.numpy as jnp
from jax import lax
from jax.experimental import pallas as pl
from jax.experimental.pallas import tpu as pltpu
from jax.experimental.pallas import tpu_sc as plsc

SELF_LOOP_ID = 38
N_NODES = 10000
N_EDGES = 320000
D = 128
N_REL = 38
N_SENT = 16
IDS_PER_SENT = 32

NC, NS = 2, 16
NW = NC * NS
LN = 16

NODES_PAD = 10240
NPT = NODES_PAD // NW
EPT = N_EDGES // NW
EPT_PAD = 10240
CH1 = 128
NPAIR = EPT_PAD // CH1 // 2
CH = 80
NODE_HALF = NODES_PAD // 2
AGG_ROWS = NODE_HALF + NS * 4
ROWS_PT = NODE_HALF // NS
WB = 80

N_TRI = N_EDGES + N_NODES
TRI_PAD = 330240
TPT = TRI_PAD // NW
NCH2 = TPT // CH
NCH2_LAST = (N_TRI - (NW - 1) * TPT) // CH

_MESH = plsc.VectorSubcoreMesh(
    core_axis_name="c", subcore_axis_name="s", num_cores=NC, num_subcores=NS)
_SC_PARAMS = pltpu.CompilerParams(needs_layout_passes=False)


def _msgpass_body(ce_hbm, cid_hbm, src_hbm, dst_hbm, rel_hbm, w_hbm,
                  relemb_hbm,
                  x0_hbm, agg_hbm,
                  cidv, src1, dst1, rel1, w1, relv, bufA, bufB,
                  gsrcA, gsrcB, dstA, dstB,
                  agg_sp, semA, semB, semSA, semSB):
    c = lax.axis_index("c")
    s = lax.axis_index("s")
    wid = c * NS + s

    pltpu.sync_copy(cid_hbm, cidv)
    eb = wid * EPT_PAD
    pltpu.sync_copy(src_hbm.at[pl.ds(eb, EPT_PAD)], src1)
    pltpu.sync_copy(dst_hbm.at[pl.ds(eb, EPT_PAD)], dst1)
    pltpu.sync_copy(rel_hbm.at[pl.ds(eb, EPT_PAD)], rel1)
    pltpu.sync_copy(w_hbm.at[pl.ds(eb, EPT_PAD)], w1)
    pltpu.sync_copy(relemb_hbm, relv)

    for q in range(NPT // 80):
        base = wid * NPT + q * 80
        pltpu.async_copy(ce_hbm.at[cidv.at[pl.ds(base, 80)]],
                         bufA.at[pl.ds(0, 80)], semA).wait()
        pltpu.sync_copy(bufA.at[pl.ds(0, 80)], x0_hbm.at[pl.ds(base, 80)])

    zv = jnp.zeros((LN,), jnp.float32)

    def zrow(r, cc):
        for j in range(D // LN):
            bufA[r, pl.ds(j * LN, LN)] = zv
        return cc

    dumpv = (lax.iota(jnp.int32, LN) & 3) + (NODE_HALF + s * 4)

    def _stage(kb, gsrcv, dstcv, p):
        for i in range(CH1 // LN):
            sv = src1[pl.ds(kb + i * LN, LN)]
            gsrcv[pl.ds(i * LN, LN)] = plsc.load_gather(cidv, [sv])
            dd = dst1[pl.ds(kb + i * LN, LN)] - (p * NODE_HALF)
            ok = (dd >= 0) & (dd < NODE_HALF)
            dstcv[pl.ds(i * LN, LN)] = jnp.where(ok, dd, dumpv)

    def _compute(kb, buf):
        def grp(g, cc):
            gb = kb + g * LN
            wvec = w1[pl.ds(gb, LN)]
            rvec = rel1[pl.ds(gb, LN)] * D
            for e16 in range(LN):
                e = g * LN + e16
                w_e = wvec[e16]
                rb = rvec[e16]
                for j in range(D // LN):
                    xv = buf[e, pl.ds(j * LN, LN)]
                    rv = relv[pl.ds(rb + j * LN, LN)]
                    buf[e, pl.ds(j * LN, LN)] = jnp.maximum(
                        xv + w_e * rv, 0.0)
            return cc

        lax.fori_loop(0, CH1 // LN, grp, 0)

    for p in range(2):
        lax.fori_loop(0, WB, zrow, 0)
        for q in range(ROWS_PT // WB):
            pltpu.sync_copy(bufA.at[pl.ds(0, WB)],
                            agg_sp.at[pl.ds(s * ROWS_PT + q * WB, WB)])
        plsc.subcore_barrier()

        def pair(m, carry):
            ka = m * 2 * CH1
            kb = ka + CH1
            _stage(ka, gsrcA, dstA, p)
            cpA = pltpu.async_copy(ce_hbm.at[gsrcA], bufA, semA)
            _stage(kb, gsrcB, dstB, p)
            cpB = pltpu.async_copy(ce_hbm.at[gsrcB], bufB, semB)
            cpA.wait()
            _compute(ka, bufA)
            scA = pltpu.async_copy(bufA, agg_sp.at[dstA], semSA, add=True)
            cpB.wait()
            _compute(kb, bufB)
            scB = pltpu.async_copy(bufB, agg_sp.at[dstB], semSB, add=True)
            scA.wait()
            scB.wait()
            return carry

        lax.fori_loop(0, NPAIR, pair, 0)
        plsc.subcore_barrier()
        for q in range(ROWS_PT // WB):
            rb = s * ROWS_PT + q * WB
            pltpu.sync_copy(agg_sp.at[pl.ds(rb, WB)], bufA.at[pl.ds(0, WB)])
            pltpu.sync_copy(
                bufA.at[pl.ds(0, WB)],
                agg_hbm.at[pl.ds((c * 2 + p) * NODE_HALF + rb, WB)])
        if p == 0:
            plsc.subcore_barrier()


_msgpass = functools.partial(
    pl.kernel,
    out_type=[
        pltpu.HBM((NODES_PAD, D), jnp.float32),
        pltpu.HBM((NC * 2 * NODE_HALF, D), jnp.float32),
    ],
    mesh=_MESH,
    scratch_types=[
        pltpu.VMEM((NODES_PAD,), jnp.int32),
        pltpu.VMEM((EPT_PAD,), jnp.int32),
        pltpu.VMEM((EPT_PAD,), jnp.int32),
        pltpu.VMEM((EPT_PAD,), jnp.int32),
        pltpu.VMEM((EPT_PAD,), jnp.float32),
        pltpu.VMEM((N_REL * D,), jnp.float32),
        pltpu.VMEM((CH1, D), jnp.float32),
        pltpu.VMEM((CH1, D), jnp.float32),
        pltpu.VMEM((CH1,), jnp.int32),
        pltpu.VMEM((CH1,), jnp.int32),
        pltpu.VMEM((CH1,), jnp.int32),
        pltpu.VMEM((CH1,), jnp.int32),
        pltpu.VMEM_SHARED((AGG_ROWS, D), jnp.float32),
        pltpu.SemaphoreType.DMA,
        pltpu.SemaphoreType.DMA,
        pltpu.SemaphoreType.DMA,
        pltpu.SemaphoreType.DMA,
    ],
    compiler_params=_SC_PARAMS,
)(_msgpass_body)


def _dense_body(x0_ref, agg_ref, W1_ref, b1_ref, W2_ref, b2_ref, Wl_ref,
                bl_ref, relsl_ref, cid2_ref, sent_ref,
                A_ref, C_ref, R2_ref, nm_ref):
    f32 = jnp.float32
    xin = (x0_ref[0:N_NODES] + agg_ref[0:N_NODES]
           + agg_ref[2 * NODE_HALF:2 * NODE_HALF + N_NODES])
    h = jnp.maximum(
        jnp.dot(xin, W1_ref[...], preferred_element_type=f32) + b1_ref[...], 0.0)
    x = jnp.dot(h, W2_ref[...], preferred_element_type=f32) + b2_ref[...]
    Wl = Wl_ref[...]
    A_ref[...] = jnp.dot(x, Wl[0:D], preferred_element_type=f32) + bl_ref[...]
    C_ref[...] = jnp.dot(x, Wl[2 * D:3 * D], preferred_element_type=f32)
    R2_ref[...] = jnp.dot(relsl_ref[...], Wl[D:2 * D], preferred_element_type=f32)
    cid2 = cid2_ref[...]
    nm = jnp.zeros_like(cid2)
    for si in range(N_SENT):
        acc = None
        for ii in range(IDS_PER_SENT):
            eq = cid2 == sent_ref[si, ii]
            acc = eq if acc is None else (acc | eq)
        nm = nm | (acc.astype(jnp.int32) << si)
    nm_ref[...] = nm


_dense = pl.pallas_call(
    _dense_body,
    out_shape=[
        jax.ShapeDtypeStruct((N_NODES, D), jnp.float32),
        jax.ShapeDtypeStruct((N_NODES, D), jnp.float32),
        jax.ShapeDtypeStruct((40, D), jnp.float32),
        jax.ShapeDtypeStruct((NODES_PAD // D, D), jnp.int32),
    ],
    in_specs=[
        pl.BlockSpec(memory_space=pltpu.VMEM),
        pl.BlockSpec(memory_space=pltpu.VMEM),
        pl.BlockSpec(memory_space=pltpu.VMEM),
        pl.BlockSpec(memory_space=pltpu.VMEM),
        pl.BlockSpec(memory_space=pltpu.VMEM),
        pl.BlockSpec(memory_space=pltpu.VMEM),
        pl.BlockSpec(memory_space=pltpu.VMEM),
        pl.BlockSpec(memory_space=pltpu.VMEM),
        pl.BlockSpec(memory_space=pltpu.VMEM),
        pl.BlockSpec(memory_space=pltpu.VMEM),
        pl.BlockSpec(memory_space=pltpu.SMEM),
    ],
)


def _asm_body(A_hbm, C_hbm, r2_hbm, cid_hbm, nm_hbm, h_hbm, t_hbm, r_hbm,
              w_hbm,
              enc_hbm, tid_hbm, mb_hbm,
              cidv, nmv, r2v, h1, t1, r1, w1, bA, bC, hc, tcv, mbb, tidf,
              semA, semB):
    c = lax.axis_index("c")
    s = lax.axis_index("s")
    wid = c * NS + s

    pltpu.sync_copy(cid_hbm, cidv)
    pltpu.sync_copy(nm_hbm, nmv)
    pltpu.sync_copy(r2_hbm, r2v)
    tb = wid * TPT
    pltpu.sync_copy(h_hbm.at[pl.ds(tb, TPT)], h1)
    pltpu.sync_copy(t_hbm.at[pl.ds(tb, TPT)], t1)
    pltpu.sync_copy(r_hbm.at[pl.ds(tb, TPT)], r1)
    pltpu.sync_copy(w_hbm.at[pl.ds(tb, TPT)], w1)

    iota3 = lax.iota(jnp.int32, LN) * 3

    def chunk(k, carry):
        kb = k * CH
        for i in range(CH // LN):
            hc[pl.ds(i * LN, LN)] = h1[pl.ds(kb + i * LN, LN)]
            tcv[pl.ds(i * LN, LN)] = t1[pl.ds(kb + i * LN, LN)]
        cpA = pltpu.async_copy(A_hbm.at[hc], bA, semA)
        cpB = pltpu.async_copy(C_hbm.at[tcv], bC, semB)
        for i in range(CH // LN):
            hv = hc[pl.ds(i * LN, LN)]
            tv = tcv[pl.ds(i * LN, LN)]
            chd = plsc.load_gather(cidv, [hv])
            ctl = plsc.load_gather(cidv, [tv])
            nh = plsc.load_gather(nmv, [hv])
            nt = plsc.load_gather(nmv, [tv])
            rv = r1[pl.ds(kb + i * LN, LN)]
            mbb[pl.ds(i * LN, LN)] = nh | nt
            base = iota3 + (3 * LN * i)
            plsc.store_scatter(tidf, [base], chd)
            plsc.store_scatter(tidf, [base + 1], rv)
            plsc.store_scatter(tidf, [base + 2], ctl)
        cpA.wait()
        cpB.wait()

        def grp(g, cc):
            gb = kb + g * LN
            wvec = w1[pl.ds(gb, LN)]
            rvec = r1[pl.ds(gb, LN)] * D
            for e16 in range(LN):
                e = g * LN + e16
                w_e = wvec[e16]
                rb = rvec[e16]
                for j in range(D // LN):
                    av = bA[e, pl.ds(j * LN, LN)]
                    cv = bC[e, pl.ds(j * LN, LN)]
                    rv2 = r2v[pl.ds(rb + j * LN, LN)]
                    bA[e, pl.ds(j * LN, LN)] = av + cv + w_e * rv2
            return cc

        lax.fori_loop(0, CH // LN, grp, 0)
        g0 = tb + kb
        pltpu.sync_copy(bA, enc_hbm.at[pl.ds(g0, CH)])
        pltpu.sync_copy(tidf, tid_hbm.at[pl.ds(g0 * 3, CH * 3)])
        pltpu.sync_copy(mbb, mb_hbm.at[pl.ds(g0, CH)])
        return carry

    nch = jnp.where(wid == NW - 1, NCH2_LAST, NCH2)
    lax.fori_loop(0, nch, chunk, 0)


_assemble = functools.partial(
    pl.kernel,
    out_type=[
        pltpu.HBM((N_TRI, D), jnp.float32),
        pltpu.HBM((N_TRI * 3,), jnp.int32),
        pltpu.HBM((N_TRI,), jnp.int32),
    ],
    mesh=_MESH,
    scratch_types=[
        pltpu.VMEM((NODES_PAD,), jnp.int32),
        pltpu.VMEM((NODES_PAD,), jnp.int32),
        pltpu.VMEM((40 * D,), jnp.float32),
        pltpu.VMEM((TPT,), jnp.int32),
        pltpu.VMEM((TPT,), jnp.int32),
        pltpu.VMEM((TPT,), jnp.int32),
        pltpu.VMEM((TPT,), jnp.float32),
        pltpu.VMEM((CH, D), jnp.float32),
        pltpu.VMEM((CH, D), jnp.float32),
        pltpu.VMEM((CH,), jnp.int32),
        pltpu.VMEM((CH,), jnp.int32),
        pltpu.VMEM((CH,), jnp.int32),
        pltpu.VMEM((CH * 3,), jnp.int32),
        pltpu.SemaphoreType.DMA,
        pltpu.SemaphoreType.DMA,
    ],
    compiler_params=_SC_PARAMS,
)(_asm_body)


def _mask_body(mb_ref, mask_ref):
    bits = mb_ref[...].reshape(1, N_TRI)
    shifts = jax.lax.broadcasted_iota(jnp.int32, (N_SENT, N_TRI), 0)
    mask_ref[...] = ((bits >> shifts) & 1) != 0


_mask_unpack = pl.pallas_call(
    _mask_body,
    out_shape=jax.ShapeDtypeStruct((N_SENT, N_TRI), jnp.bool_),
)


_EB = 13200


def _strip_body(enc_in, tid_in, enc_out, tid_out):
    enc_out[...] = enc_in[...]
    tid_out[...] = tid_in[...]


_strip = pl.pallas_call(
    _strip_body,
    grid=(N_TRI // _EB,),
    in_specs=[
        pl.BlockSpec((_EB, D), lambda i: (i, 0)),
        pl.BlockSpec((_EB, 3), lambda i: (i, 0)),
    ],
    out_specs=[
        pl.BlockSpec((_EB, D), lambda i: (i, 0)),
        pl.BlockSpec((_EB, 3), lambda i: (i, 0)),
    ],
    out_shape=[
        jax.ShapeDtypeStruct((N_TRI, D), jnp.float32),
        jax.ShapeDtypeStruct((N_TRI, 3), jnp.int32),
    ],
)


def kernel(concept_ids, edge_index, edge_relation, edge_weight,
           sent_concept_ids, concept_embedding, relation_embedding,
           self_loop_embedding, W1, b1, W2, b2, W_lin, b_lin):
    i32, f32 = jnp.int32, jnp.float32
    cid = concept_ids.astype(i32)
    src = edge_index[0].astype(i32)
    dst = edge_index[1].astype(i32)
    rel = edge_relation.astype(i32)
    w = edge_weight.astype(f32)

    cid_pad = jnp.concatenate([cid, jnp.zeros((NODES_PAD - N_NODES,), i32)])
    relv_flat = relation_embedding.astype(f32).reshape(-1)

    epad = NW * EPT_PAD - N_EDGES
    src_p = jnp.concatenate([src, jnp.zeros((epad,), i32)])
    dst_p = jnp.concatenate([dst, jnp.full((epad,), NODES_PAD, i32)])
    rel_p = jnp.concatenate([rel, jnp.zeros((epad,), i32)])
    w_p = jnp.concatenate([w, jnp.zeros((epad,), f32)])
    x0h, aggf = _msgpass(concept_embedding.astype(f32), cid_pad, src_p,
                         dst_p, rel_p, w_p, relv_flat)

    relsl = jnp.concatenate(
        [relation_embedding.astype(f32), self_loop_embedding.astype(f32),
         jnp.zeros((1, D), f32)], axis=0)
    cid2d = cid_pad.reshape(NODES_PAD // D, D)
    A, C, R2, nm2d = _dense(
        x0h, aggf, W1.astype(f32), b1.astype(f32).reshape(1, D),
        W2.astype(f32), b2.astype(f32).reshape(1, D), W_lin.astype(f32),
        b_lin.astype(f32).reshape(1, D), relsl, cid2d,
        sent_concept_ids.astype(i32))

    nm_flat = nm2d.reshape(-1)
    ar_n = jnp.arange(N_NODES, dtype=i32)
    npad = TRI_PAD - N_TRI
    h_ext = jnp.concatenate([src, ar_n, jnp.zeros((npad,), i32)])
    t_ext = jnp.concatenate([dst, ar_n, jnp.zeros((npad,), i32)])
    r_ext = jnp.concatenate(
        [rel, jnp.full((N_NODES,), SELF_LOOP_ID, i32), jnp.zeros((npad,), i32)])
    w_ext = jnp.concatenate(
        [w, jnp.ones((N_NODES,), f32), jnp.zeros((npad,), f32)])

    enc, tid_flat, mb = _assemble(
        A, C, R2.reshape(-1), cid_pad, nm_flat, h_ext, t_ext, r_ext, w_ext)

    mask = _mask_unpack(mb)
    enc, tid = _strip(enc, tid_flat.reshape(N_TRI, 3))
    return enc, mask, tid


# speedup vs baseline: 1.1983x; 1.1955x over previous
import functools

import jax
import jax.numpy as jnp
from jax import lax
from jax.experimental import pallas as pl
from jax.experimental.pallas import tpu as pltpu
from jax.experimental.pallas import tpu_sc as plsc

SELF_LOOP_ID = 38
N_NODES = 10000
N_EDGES = 320000
D = 128
N_REL = 38
N_SENT = 16
IDS_PER_SENT = 32

NC, NS = 2, 16
NW = NC * NS
LN = 16

NODES_PAD = 10240
NPT = NODES_PAD // NW
EPT = N_EDGES // NW
EPT_PAD = 10240
CH1 = 128
NPAIR = EPT_PAD // CH1 // 2
CH = 80
NODE_HALF = NODES_PAD // 2
AGG_ROWS = NODE_HALF + NS * 4
ROWS_PT = NODE_HALF // NS
WB = 80

N_TRI = N_EDGES + N_NODES
TRI_PAD = 330240
TPT = TRI_PAD // NW
NCH2 = TPT // CH
NCH2_LAST = (N_TRI - (NW - 1) * TPT) // CH

_MESH = plsc.VectorSubcoreMesh(
    core_axis_name="c", subcore_axis_name="s", num_cores=NC, num_subcores=NS)
_SC_PARAMS = pltpu.CompilerParams(needs_layout_passes=False)


def _msgpass_body(ce_hbm, cid_hbm, src_hbm, dst_hbm, rel_hbm, w_hbm,
                  relemb_hbm,
                  x0_hbm, agg_hbm,
                  cidv, src1, dst1, rel1, w1, relv, bufA, bufB,
                  gsrcA, gsrcB, dstA, dstB,
                  agg_sp, semA, semB, semSA, semSB):
    c = lax.axis_index("c")
    s = lax.axis_index("s")
    wid = c * NS + s

    pltpu.sync_copy(cid_hbm, cidv)
    eb = wid * EPT_PAD
    pltpu.sync_copy(src_hbm.at[pl.ds(eb, EPT_PAD)], src1)
    pltpu.sync_copy(dst_hbm.at[pl.ds(eb, EPT_PAD)], dst1)
    pltpu.sync_copy(rel_hbm.at[pl.ds(eb, EPT_PAD)], rel1)
    pltpu.sync_copy(w_hbm.at[pl.ds(eb, EPT_PAD)], w1)
    pltpu.sync_copy(relemb_hbm, relv)

    for q in range(NPT // 80):
        base = wid * NPT + q * 80
        pltpu.async_copy(ce_hbm.at[cidv.at[pl.ds(base, 80)]],
                         bufA.at[pl.ds(0, 80)], semA).wait()
        pltpu.sync_copy(bufA.at[pl.ds(0, 80)], x0_hbm.at[pl.ds(base, 80)])

    zv = jnp.zeros((LN,), jnp.float32)

    def zrow(r, cc):
        for j in range(D // LN):
            bufA[r, pl.ds(j * LN, LN)] = zv
        return cc

    dumpv = (lax.iota(jnp.int32, LN) & 3) + (NODE_HALF + s * 4)

    def _stage(kb, gsrcv, dstcv, p):
        for i in range(CH1 // LN):
            sv = src1[pl.ds(kb + i * LN, LN)]
            gsrcv[pl.ds(i * LN, LN)] = plsc.load_gather(cidv, [sv])
            dd = dst1[pl.ds(kb + i * LN, LN)] - (p * NODE_HALF)
            ok = (dd >= 0) & (dd < NODE_HALF)
            dstcv[pl.ds(i * LN, LN)] = jnp.where(ok, dd, dumpv)

    def _compute(kb, buf):
        def grp(g, cc):
            gb = kb + g * LN
            wvec = w1[pl.ds(gb, LN)]
            rvec = rel1[pl.ds(gb, LN)] * D
            for e16 in range(LN):
                e = g * LN + e16
                w_e = wvec[e16]
                rb = rvec[e16]
                for j in range(D // LN):
                    xv = buf[e, pl.ds(j * LN, LN)]
                    rv = relv[pl.ds(rb + j * LN, LN)]
                    buf[e, pl.ds(j * LN, LN)] = jnp.maximum(
                        xv + w_e * rv, 0.0)
            return cc

        lax.fori_loop(0, CH1 // LN, grp, 0)

    for p in range(2):
        lax.fori_loop(0, WB, zrow, 0)
        for q in range(ROWS_PT // WB):
            pltpu.sync_copy(bufA.at[pl.ds(0, WB)],
                            agg_sp.at[pl.ds(s * ROWS_PT + q * WB, WB)])
        plsc.subcore_barrier()

        def pair(m, carry):
            ka = m * 2 * CH1
            kb = ka + CH1
            _stage(ka, gsrcA, dstA, p)
            cpA = pltpu.async_copy(ce_hbm.at[gsrcA], bufA, semA)
            _stage(kb, gsrcB, dstB, p)
            cpB = pltpu.async_copy(ce_hbm.at[gsrcB], bufB, semB)
            cpA.wait()
            _compute(ka, bufA)
            scA = pltpu.async_copy(bufA, agg_sp.at[dstA], semSA, add=True)
            cpB.wait()
            _compute(kb, bufB)
            scB = pltpu.async_copy(bufB, agg_sp.at[dstB], semSB, add=True)
            scA.wait()
            scB.wait()
            return carry

        lax.fori_loop(0, NPAIR, pair, 0)
        plsc.subcore_barrier()
        for q in range(ROWS_PT // WB):
            rb = s * ROWS_PT + q * WB
            pltpu.sync_copy(agg_sp.at[pl.ds(rb, WB)], bufA.at[pl.ds(0, WB)])
            pltpu.sync_copy(
                bufA.at[pl.ds(0, WB)],
                agg_hbm.at[pl.ds((c * 2 + p) * NODE_HALF + rb, WB)])
        if p == 0:
            plsc.subcore_barrier()


_msgpass = functools.partial(
    pl.kernel,
    out_type=[
        pltpu.HBM((NODES_PAD, D), jnp.float32),
        pltpu.HBM((NC * 2 * NODE_HALF, D), jnp.float32),
    ],
    mesh=_MESH,
    scratch_types=[
        pltpu.VMEM((NODES_PAD,), jnp.int32),
        pltpu.VMEM((EPT_PAD,), jnp.int32),
        pltpu.VMEM((EPT_PAD,), jnp.int32),
        pltpu.VMEM((EPT_PAD,), jnp.int32),
        pltpu.VMEM((EPT_PAD,), jnp.float32),
        pltpu.VMEM((N_REL * D,), jnp.float32),
        pltpu.VMEM((CH1, D), jnp.float32),
        pltpu.VMEM((CH1, D), jnp.float32),
        pltpu.VMEM((CH1,), jnp.int32),
        pltpu.VMEM((CH1,), jnp.int32),
        pltpu.VMEM((CH1,), jnp.int32),
        pltpu.VMEM((CH1,), jnp.int32),
        pltpu.VMEM_SHARED((AGG_ROWS, D), jnp.float32),
        pltpu.SemaphoreType.DMA,
        pltpu.SemaphoreType.DMA,
        pltpu.SemaphoreType.DMA,
        pltpu.SemaphoreType.DMA,
    ],
    compiler_params=_SC_PARAMS,
)(_msgpass_body)


def _dense_body(x0_ref, agg_ref, W1_ref, b1_ref, W2_ref, b2_ref, Wl_ref,
                bl_ref, relsl_ref, cid2_ref, sent_ref,
                A_ref, C_ref, R2_ref, nm_ref):
    f32 = jnp.float32
    xin = (x0_ref[0:N_NODES] + agg_ref[0:N_NODES]
           + agg_ref[2 * NODE_HALF:2 * NODE_HALF + N_NODES])
    h = jnp.maximum(
        jnp.dot(xin, W1_ref[...], preferred_element_type=f32) + b1_ref[...], 0.0)
    x = jnp.dot(h, W2_ref[...], preferred_element_type=f32) + b2_ref[...]
    Wl = Wl_ref[...]
    A_ref[...] = jnp.dot(x, Wl[0:D], preferred_element_type=f32) + bl_ref[...]
    C_ref[...] = jnp.dot(x, Wl[2 * D:3 * D], preferred_element_type=f32)
    R2_ref[...] = jnp.dot(relsl_ref[...], Wl[D:2 * D], preferred_element_type=f32)
    cid2 = cid2_ref[...]
    nm = jnp.zeros_like(cid2)
    for si in range(N_SENT):
        acc = None
        for ii in range(IDS_PER_SENT):
            eq = cid2 == sent_ref[si, ii]
            acc = eq if acc is None else (acc | eq)
        nm = nm | (acc.astype(jnp.int32) << si)
    nm_ref[...] = nm


_dense = pl.pallas_call(
    _dense_body,
    out_shape=[
        jax.ShapeDtypeStruct((N_NODES, D), jnp.float32),
        jax.ShapeDtypeStruct((N_NODES, D), jnp.float32),
        jax.ShapeDtypeStruct((40, D), jnp.float32),
        jax.ShapeDtypeStruct((NODES_PAD // D, D), jnp.int32),
    ],
    in_specs=[
        pl.BlockSpec(memory_space=pltpu.VMEM),
        pl.BlockSpec(memory_space=pltpu.VMEM),
        pl.BlockSpec(memory_space=pltpu.VMEM),
        pl.BlockSpec(memory_space=pltpu.VMEM),
        pl.BlockSpec(memory_space=pltpu.VMEM),
        pl.BlockSpec(memory_space=pltpu.VMEM),
        pl.BlockSpec(memory_space=pltpu.VMEM),
        pl.BlockSpec(memory_space=pltpu.VMEM),
        pl.BlockSpec(memory_space=pltpu.VMEM),
        pl.BlockSpec(memory_space=pltpu.VMEM),
        pl.BlockSpec(memory_space=pltpu.SMEM),
    ],
)


def _asm_body(A_hbm, C_hbm, r2_hbm, cid_hbm, nm_hbm, h_hbm, t_hbm, r_hbm,
              w_hbm,
              enc_hbm, tid_hbm, mb_hbm,
              cidv, nmv, r2v, h1, t1, r1, w1, bA, bC, hc, tcv, mbb, tidf,
              semA, semB):
    c = lax.axis_index("c")
    s = lax.axis_index("s")
    wid = c * NS + s

    pltpu.sync_copy(cid_hbm, cidv)
    pltpu.sync_copy(nm_hbm, nmv)
    pltpu.sync_copy(r2_hbm, r2v)
    tb = wid * TPT
    pltpu.sync_copy(h_hbm.at[pl.ds(tb, TPT)], h1)
    pltpu.sync_copy(t_hbm.at[pl.ds(tb, TPT)], t1)
    pltpu.sync_copy(r_hbm.at[pl.ds(tb, TPT)], r1)
    pltpu.sync_copy(w_hbm.at[pl.ds(tb, TPT)], w1)

    iota3 = lax.iota(jnp.int32, LN) * 3

    def chunk(k, carry):
        kb = k * CH
        for i in range(CH // LN):
            hc[pl.ds(i * LN, LN)] = h1[pl.ds(kb + i * LN, LN)]
            tcv[pl.ds(i * LN, LN)] = t1[pl.ds(kb + i * LN, LN)]
        cpA = pltpu.async_copy(A_hbm.at[hc], bA, semA)
        cpB = pltpu.async_copy(C_hbm.at[tcv], bC, semB)
        for i in range(CH // LN):
            hv = hc[pl.ds(i * LN, LN)]
            tv = tcv[pl.ds(i * LN, LN)]
            chd = plsc.load_gather(cidv, [hv])
            ctl = plsc.load_gather(cidv, [tv])
            nh = plsc.load_gather(nmv, [hv])
            nt = plsc.load_gather(nmv, [tv])
            rv = r1[pl.ds(kb + i * LN, LN)]
            mbb[pl.ds(i * LN, LN)] = nh | nt
            base = iota3 + (3 * LN * i)
            plsc.store_scatter(tidf, [base], chd)
            plsc.store_scatter(tidf, [base + 1], rv)
            plsc.store_scatter(tidf, [base + 2], ctl)
        cpA.wait()
        cpB.wait()

        def grp(g, cc):
            gb = kb + g * LN
            wvec = w1[pl.ds(gb, LN)]
            rvec = r1[pl.ds(gb, LN)] * D
            for e16 in range(LN):
                e = g * LN + e16
                w_e = wvec[e16]
                rb = rvec[e16]
                for j in range(D // LN):
                    av = bA[e, pl.ds(j * LN, LN)]
                    cv = bC[e, pl.ds(j * LN, LN)]
                    rv2 = r2v[pl.ds(rb + j * LN, LN)]
                    bA[e, pl.ds(j * LN, LN)] = av + cv + w_e * rv2
            return cc

        lax.fori_loop(0, CH // LN, grp, 0)
        g0 = tb + kb
        pltpu.sync_copy(bA, enc_hbm.at[pl.ds(g0, CH)])
        pltpu.sync_copy(tidf, tid_hbm.at[pl.ds(g0 * 3, CH * 3)])
        pltpu.sync_copy(mbb, mb_hbm.at[pl.ds(g0, CH)])
        return carry

    nch = jnp.where(wid == NW - 1, NCH2_LAST, NCH2)
    lax.fori_loop(0, nch, chunk, 0)


_assemble = functools.partial(
    pl.kernel,
    out_type=[
        pltpu.HBM((N_TRI, D), jnp.float32),
        pltpu.HBM((N_TRI * 3,), jnp.int32),
        pltpu.HBM((N_TRI,), jnp.int32),
    ],
    mesh=_MESH,
    scratch_types=[
        pltpu.VMEM((NODES_PAD,), jnp.int32),
        pltpu.VMEM((NODES_PAD,), jnp.int32),
        pltpu.VMEM((40 * D,), jnp.float32),
        pltpu.VMEM((TPT,), jnp.int32),
        pltpu.VMEM((TPT,), jnp.int32),
        pltpu.VMEM((TPT,), jnp.int32),
        pltpu.VMEM((TPT,), jnp.float32),
        pltpu.VMEM((CH, D), jnp.float32),
        pltpu.VMEM((CH, D), jnp.float32),
        pltpu.VMEM((CH,), jnp.int32),
        pltpu.VMEM((CH,), jnp.int32),
        pltpu.VMEM((CH,), jnp.int32),
        pltpu.VMEM((CH * 3,), jnp.int32),
        pltpu.SemaphoreType.DMA,
        pltpu.SemaphoreType.DMA,
    ],
    compiler_params=_SC_PARAMS,
)(_asm_body)


def _mask_body(mb_ref, mask_ref):
    bits = mb_ref[...].reshape(1, N_TRI)
    shifts = jax.lax.broadcasted_iota(jnp.int32, (N_SENT, N_TRI), 0)
    mask_ref[...] = ((bits >> shifts) & 1) != 0


_mask_unpack = pl.pallas_call(
    _mask_body,
    out_shape=jax.ShapeDtypeStruct((N_SENT, N_TRI), jnp.bool_),
)


_EB = 13200


def _strip_body(enc_in, tid_in, enc_out, tid_out):
    enc_out[...] = enc_in[...]
    tid_out[...] = tid_in[...]


_strip = pl.pallas_call(
    _strip_body,
    grid=(N_TRI // _EB,),
    in_specs=[
        pl.BlockSpec((_EB, D), lambda i: (i, 0)),
        pl.BlockSpec((_EB, 3), lambda i: (i, 0)),
    ],
    out_specs=[
        pl.BlockSpec((_EB, D), lambda i: (i, 0)),
        pl.BlockSpec((_EB, 3), lambda i: (i, 0)),
    ],
    out_shape=[
        jax.ShapeDtypeStruct((N_TRI, D), jnp.float32),
        jax.ShapeDtypeStruct((N_TRI, 3), jnp.int32),
    ],
)


def kernel(concept_ids, edge_index, edge_relation, edge_weight,
           sent_concept_ids, concept_embedding, relation_embedding,
           self_loop_embedding, W1, b1, W2, b2, W_lin, b_lin):
    i32, f32 = jnp.int32, jnp.float32
    cid = concept_ids.astype(i32)
    src = edge_index[0].astype(i32)
    dst = edge_index[1].astype(i32)
    rel = edge_relation.astype(i32)
    w = edge_weight.astype(f32)

    npad_n = NODES_PAD - N_NODES
    vocab = concept_embedding.shape[0]
    cid_pad = jnp.concatenate([cid, vocab + jnp.arange(npad_n, dtype=i32)])
    ce_aug = jnp.concatenate(
        [concept_embedding.astype(f32), jnp.zeros((npad_n, D), f32)], axis=0)
    relv_flat = relation_embedding.astype(f32).reshape(-1)

    epad = NW * EPT_PAD - N_EDGES
    ar_e = jnp.arange(epad, dtype=i32)
    src_p = jnp.concatenate([src, N_NODES + (ar_e % npad_n)])
    dst_p = jnp.concatenate([dst, ar_e % N_NODES])
    rel_p = jnp.concatenate([rel, jnp.zeros((epad,), i32)])
    w_p = jnp.concatenate([w, jnp.zeros((epad,), f32)])
    x0h, aggf = _msgpass(ce_aug, cid_pad, src_p, dst_p, rel_p, w_p, relv_flat)

    relsl = jnp.concatenate(
        [relation_embedding.astype(f32), self_loop_embedding.astype(f32),
         jnp.zeros((1, D), f32)], axis=0)
    cid2d = cid_pad.reshape(NODES_PAD // D, D)
    A, C, R2, nm2d = _dense(
        x0h, aggf, W1.astype(f32), b1.astype(f32).reshape(1, D),
        W2.astype(f32), b2.astype(f32).reshape(1, D), W_lin.astype(f32),
        b_lin.astype(f32).reshape(1, D), relsl, cid2d,
        sent_concept_ids.astype(i32))

    nm_flat = nm2d.reshape(-1)
    ar_n = jnp.arange(N_NODES, dtype=i32)
    npad = TRI_PAD - N_TRI
    h_ext = jnp.concatenate([src, ar_n, jnp.zeros((npad,), i32)])
    t_ext = jnp.concatenate([dst, ar_n, jnp.zeros((npad,), i32)])
    r_ext = jnp.concatenate(
        [rel, jnp.full((N_NODES,), SELF_LOOP_ID, i32), jnp.zeros((npad,), i32)])
    w_ext = jnp.concatenate(
        [w, jnp.ones((N_NODES,), f32), jnp.zeros((npad,), f32)])

    enc, tid_flat, mb = _assemble(
        A, C, R2.reshape(-1), cid_pad, nm_flat, h_ext, t_ext, r_ext, w_ext)

    mask = _mask_unpack(mb)
    enc, tid = _strip(enc, tid_flat.reshape(N_TRI, 3))
    return enc, mask, tid


# speedup vs baseline: 1.2266x; 1.0236x over previous
import functools

import jax
import jax.numpy as jnp
from jax import lax
from jax.experimental import pallas as pl
from jax.experimental.pallas import tpu as pltpu
from jax.experimental.pallas import tpu_sc as plsc

SELF_LOOP_ID = 38
N_NODES = 10000
N_EDGES = 320000
D = 128
N_REL = 38
N_SENT = 16
IDS_PER_SENT = 32

NC, NS = 2, 16
NW = NC * NS
LN = 16

NODES_PAD = 10240
NPT = NODES_PAD // NW
EPT = N_EDGES // NW
EPT_PAD = 10240
CH1 = 128
NPAIR = EPT_PAD // CH1 // 2
CH = 80
NODE_HALF = NODES_PAD // 2
AGG_ROWS = NODE_HALF + NS * 4
ROWS_PT = NODE_HALF // NS
WB = 80

N_TRI = N_EDGES + N_NODES
TRI_PAD = 332800
TPT = TRI_PAD // NW
NPAIR2 = TPT // CH // 2

_MESH = plsc.VectorSubcoreMesh(
    core_axis_name="c", subcore_axis_name="s", num_cores=NC, num_subcores=NS)
_SC_PARAMS = pltpu.CompilerParams(needs_layout_passes=False)


def _msgpass_body(ce_hbm, cid_hbm, src_hbm, dst_hbm, rel_hbm, w_hbm,
                  relemb_hbm,
                  x0_hbm, agg_hbm,
                  cidv, src1, dst1, rel1, w1, relv, bufA, bufB,
                  gsrcA, gsrcB, dstA, dstB,
                  agg_sp, semA, semB, semSA, semSB):
    c = lax.axis_index("c")
    s = lax.axis_index("s")
    wid = c * NS + s

    pltpu.sync_copy(cid_hbm, cidv)
    eb = wid * EPT_PAD
    pltpu.sync_copy(src_hbm.at[pl.ds(eb, EPT_PAD)], src1)
    pltpu.sync_copy(dst_hbm.at[pl.ds(eb, EPT_PAD)], dst1)
    pltpu.sync_copy(rel_hbm.at[pl.ds(eb, EPT_PAD)], rel1)
    pltpu.sync_copy(w_hbm.at[pl.ds(eb, EPT_PAD)], w1)
    pltpu.sync_copy(relemb_hbm, relv)

    for q in range(NPT // 80):
        base = wid * NPT + q * 80
        pltpu.async_copy(ce_hbm.at[cidv.at[pl.ds(base, 80)]],
                         bufA.at[pl.ds(0, 80)], semA).wait()
        pltpu.sync_copy(bufA.at[pl.ds(0, 80)], x0_hbm.at[pl.ds(base, 80)])

    zv = jnp.zeros((LN,), jnp.float32)

    def zrow(r, cc):
        for j in range(D // LN):
            bufA[r, pl.ds(j * LN, LN)] = zv
        return cc

    dumpv = (lax.iota(jnp.int32, LN) & 3) + (NODE_HALF + s * 4)

    def _stage(kb, gsrcv, dstcv, p):
        for i in range(CH1 // LN):
            sv = src1[pl.ds(kb + i * LN, LN)]
            gsrcv[pl.ds(i * LN, LN)] = plsc.load_gather(cidv, [sv])
            dd = dst1[pl.ds(kb + i * LN, LN)] - (p * NODE_HALF)
            ok = (dd >= 0) & (dd < NODE_HALF)
            dstcv[pl.ds(i * LN, LN)] = jnp.where(ok, dd, dumpv)

    def _compute(kb, buf):
        def grp(g, cc):
            gb = kb + g * LN
            wvec = w1[pl.ds(gb, LN)]
            rvec = rel1[pl.ds(gb, LN)] * D
            for e16 in range(LN):
                e = g * LN + e16
                w_e = wvec[e16]
                rb = rvec[e16]
                for j in range(D // LN):
                    xv = buf[e, pl.ds(j * LN, LN)]
                    rv = relv[pl.ds(rb + j * LN, LN)]
                    buf[e, pl.ds(j * LN, LN)] = jnp.maximum(
                        xv + w_e * rv, 0.0)
            return cc

        lax.fori_loop(0, CH1 // LN, grp, 0)

    for p in range(2):
        lax.fori_loop(0, WB, zrow, 0)
        for q in range(ROWS_PT // WB):
            pltpu.sync_copy(bufA.at[pl.ds(0, WB)],
                            agg_sp.at[pl.ds(s * ROWS_PT + q * WB, WB)])
        plsc.subcore_barrier()

        def pair(m, carry):
            ka = m * 2 * CH1
            kb = ka + CH1
            _stage(ka, gsrcA, dstA, p)
            cpA = pltpu.async_copy(ce_hbm.at[gsrcA], bufA, semA)
            _stage(kb, gsrcB, dstB, p)
            cpB = pltpu.async_copy(ce_hbm.at[gsrcB], bufB, semB)
            cpA.wait()
            _compute(ka, bufA)
            scA = pltpu.async_copy(bufA, agg_sp.at[dstA], semSA, add=True)
            cpB.wait()
            _compute(kb, bufB)
            scB = pltpu.async_copy(bufB, agg_sp.at[dstB], semSB, add=True)
            scA.wait()
            scB.wait()
            return carry

        lax.fori_loop(0, NPAIR, pair, 0)
        plsc.subcore_barrier()
        for q in range(ROWS_PT // WB):
            rb = s * ROWS_PT + q * WB
            pltpu.sync_copy(agg_sp.at[pl.ds(rb, WB)], bufA.at[pl.ds(0, WB)])
            pltpu.sync_copy(
                bufA.at[pl.ds(0, WB)],
                agg_hbm.at[pl.ds((c * 2 + p) * NODE_HALF + rb, WB)])
        if p == 0:
            plsc.subcore_barrier()


_msgpass = functools.partial(
    pl.kernel,
    out_type=[
        pltpu.HBM((NODES_PAD, D), jnp.float32),
        pltpu.HBM((NC * 2 * NODE_HALF, D), jnp.float32),
    ],
    mesh=_MESH,
    scratch_types=[
        pltpu.VMEM((NODES_PAD,), jnp.int32),
        pltpu.VMEM((EPT_PAD,), jnp.int32),
        pltpu.VMEM((EPT_PAD,), jnp.int32),
        pltpu.VMEM((EPT_PAD,), jnp.int32),
        pltpu.VMEM((EPT_PAD,), jnp.float32),
        pltpu.VMEM((N_REL * D,), jnp.float32),
        pltpu.VMEM((CH1, D), jnp.float32),
        pltpu.VMEM((CH1, D), jnp.float32),
        pltpu.VMEM((CH1,), jnp.int32),
        pltpu.VMEM((CH1,), jnp.int32),
        pltpu.VMEM((CH1,), jnp.int32),
        pltpu.VMEM((CH1,), jnp.int32),
        pltpu.VMEM_SHARED((AGG_ROWS, D), jnp.float32),
        pltpu.SemaphoreType.DMA,
        pltpu.SemaphoreType.DMA,
        pltpu.SemaphoreType.DMA,
        pltpu.SemaphoreType.DMA,
    ],
    compiler_params=_SC_PARAMS,
)(_msgpass_body)


def _dense_body(x0_ref, agg_ref, W1_ref, b1_ref, W2_ref, b2_ref, Wl_ref,
                bl_ref, relsl_ref, cid2_ref, sent_ref,
                A_ref, C_ref, R2_ref, nm_ref):
    f32 = jnp.float32
    xin = (x0_ref[0:N_NODES] + agg_ref[0:N_NODES]
           + agg_ref[2 * NODE_HALF:2 * NODE_HALF + N_NODES])
    h = jnp.maximum(
        jnp.dot(xin, W1_ref[...], preferred_element_type=f32) + b1_ref[...], 0.0)
    x = jnp.dot(h, W2_ref[...], preferred_element_type=f32) + b2_ref[...]
    Wl = Wl_ref[...]
    A_ref[...] = jnp.dot(x, Wl[0:D], preferred_element_type=f32) + bl_ref[...]
    C_ref[...] = jnp.dot(x, Wl[2 * D:3 * D], preferred_element_type=f32)
    R2_ref[...] = jnp.dot(relsl_ref[...], Wl[D:2 * D], preferred_element_type=f32)
    cid2 = cid2_ref[...]
    nm = jnp.zeros_like(cid2)
    for si in range(N_SENT):
        acc = None
        for ii in range(IDS_PER_SENT):
            eq = cid2 == sent_ref[si, ii]
            acc = eq if acc is None else (acc | eq)
        nm = nm | (acc.astype(jnp.int32) << si)
    nm_ref[...] = nm


_dense = pl.pallas_call(
    _dense_body,
    out_shape=[
        jax.ShapeDtypeStruct((N_NODES, D), jnp.float32),
        jax.ShapeDtypeStruct((N_NODES, D), jnp.float32),
        jax.ShapeDtypeStruct((40, D), jnp.float32),
        jax.ShapeDtypeStruct((NODES_PAD // D, D), jnp.int32),
    ],
    in_specs=[
        pl.BlockSpec(memory_space=pltpu.VMEM),
        pl.BlockSpec(memory_space=pltpu.VMEM),
        pl.BlockSpec(memory_space=pltpu.VMEM),
        pl.BlockSpec(memory_space=pltpu.VMEM),
        pl.BlockSpec(memory_space=pltpu.VMEM),
        pl.BlockSpec(memory_space=pltpu.VMEM),
        pl.BlockSpec(memory_space=pltpu.VMEM),
        pl.BlockSpec(memory_space=pltpu.VMEM),
        pl.BlockSpec(memory_space=pltpu.VMEM),
        pl.BlockSpec(memory_space=pltpu.VMEM),
        pl.BlockSpec(memory_space=pltpu.SMEM),
    ],
)


def _asm_body(A_hbm, C_hbm, r2_hbm, cid_hbm, nm_hbm, h_hbm, t_hbm, r_hbm,
              w_hbm,
              enc_hbm, tid_hbm, mb_hbm,
              cidv, nmv, r2v, h1, t1, r1, w1,
              bA0, bC0, bA1, bC1, hc0, tc0, hc1, tc1,
              mbb0, tidf0, mbb1, tidf1,
              semA0, semC0, semA1, semC1, semW0, semW1):
    c = lax.axis_index("c")
    s = lax.axis_index("s")
    wid = c * NS + s

    pltpu.sync_copy(cid_hbm, cidv)
    pltpu.sync_copy(nm_hbm, nmv)
    pltpu.sync_copy(r2_hbm, r2v)
    tb = wid * TPT
    pltpu.sync_copy(h_hbm.at[pl.ds(tb, TPT)], h1)
    pltpu.sync_copy(t_hbm.at[pl.ds(tb, TPT)], t1)
    pltpu.sync_copy(r_hbm.at[pl.ds(tb, TPT)], r1)
    pltpu.sync_copy(w_hbm.at[pl.ds(tb, TPT)], w1)

    iota3 = lax.iota(jnp.int32, LN) * 3

    def _stage(kb, hc, tcv):
        for i in range(CH // LN):
            hc[pl.ds(i * LN, LN)] = h1[pl.ds(kb + i * LN, LN)]
            tcv[pl.ds(i * LN, LN)] = t1[pl.ds(kb + i * LN, LN)]

    def _ids(kb, hc, tcv, mbb, tidf):
        for i in range(CH // LN):
            hv = hc[pl.ds(i * LN, LN)]
            tv = tcv[pl.ds(i * LN, LN)]
            chd = plsc.load_gather(cidv, [hv])
            ctl = plsc.load_gather(cidv, [tv])
            nh = plsc.load_gather(nmv, [hv])
            nt = plsc.load_gather(nmv, [tv])
            rv = r1[pl.ds(kb + i * LN, LN)]
            mbb[pl.ds(i * LN, LN)] = nh | nt
            base = iota3 + (3 * LN * i)
            plsc.store_scatter(tidf, [base], chd)
            plsc.store_scatter(tidf, [base + 1], rv)
            plsc.store_scatter(tidf, [base + 2], ctl)

    def _compute(kb, bA, bC):
        def grp(g, cc):
            gb = kb + g * LN
            wvec = w1[pl.ds(gb, LN)]
            rvec = r1[pl.ds(gb, LN)] * D
            for e16 in range(LN):
                e = g * LN + e16
                w_e = wvec[e16]
                rb = rvec[e16]
                for j in range(D // LN):
                    av = bA[e, pl.ds(j * LN, LN)]
                    cv = bC[e, pl.ds(j * LN, LN)]
                    rv2 = r2v[pl.ds(rb + j * LN, LN)]
                    bA[e, pl.ds(j * LN, LN)] = av + cv + w_e * rv2
            return cc

        lax.fori_loop(0, CH // LN, grp, 0)

    def pair(m, carry):
        ka = m * 2 * CH
        kb = ka + CH
        _stage(ka, hc0, tc0)
        cpA0 = pltpu.async_copy(A_hbm.at[hc0], bA0, semA0)
        cpC0 = pltpu.async_copy(C_hbm.at[tc0], bC0, semC0)
        _stage(kb, hc1, tc1)
        cpA1 = pltpu.async_copy(A_hbm.at[hc1], bA1, semA1)
        cpC1 = pltpu.async_copy(C_hbm.at[tc1], bC1, semC1)
        _ids(ka, hc0, tc0, mbb0, tidf0)
        cpA0.wait()
        cpC0.wait()
        _compute(ka, bA0, bC0)
        w0 = pltpu.async_copy(bA0, enc_hbm.at[pl.ds(tb + ka, CH)], semW0)
        _ids(kb, hc1, tc1, mbb1, tidf1)
        cpA1.wait()
        cpC1.wait()
        _compute(kb, bA1, bC1)
        w1c = pltpu.async_copy(bA1, enc_hbm.at[pl.ds(tb + kb, CH)], semW1)
        g0 = tb + ka
        pltpu.sync_copy(tidf0, tid_hbm.at[pl.ds(g0 * 3, CH * 3)])
        pltpu.sync_copy(mbb0, mb_hbm.at[pl.ds(g0, CH)])
        g1 = tb + kb
        pltpu.sync_copy(tidf1, tid_hbm.at[pl.ds(g1 * 3, CH * 3)])
        pltpu.sync_copy(mbb1, mb_hbm.at[pl.ds(g1, CH)])
        w0.wait()
        w1c.wait()
        return carry

    lax.fori_loop(0, NPAIR2, pair, 0)


_assemble = functools.partial(
    pl.kernel,
    out_type=[
        pltpu.HBM((TRI_PAD, D), jnp.float32),
        pltpu.HBM((TRI_PAD * 3,), jnp.int32),
        pltpu.HBM((TRI_PAD,), jnp.int32),
    ],
    mesh=_MESH,
    scratch_types=[
        pltpu.VMEM((NODES_PAD,), jnp.int32),
        pltpu.VMEM((NODES_PAD,), jnp.int32),
        pltpu.VMEM((40 * D,), jnp.float32),
        pltpu.VMEM((TPT,), jnp.int32),
        pltpu.VMEM((TPT,), jnp.int32),
        pltpu.VMEM((TPT,), jnp.int32),
        pltpu.VMEM((TPT,), jnp.float32),
        pltpu.VMEM((CH, D), jnp.float32),
        pltpu.VMEM((CH, D), jnp.float32),
        pltpu.VMEM((CH, D), jnp.float32),
        pltpu.VMEM((CH, D), jnp.float32),
        pltpu.VMEM((CH,), jnp.int32),
        pltpu.VMEM((CH,), jnp.int32),
        pltpu.VMEM((CH,), jnp.int32),
        pltpu.VMEM((CH,), jnp.int32),
        pltpu.VMEM((CH,), jnp.int32),
        pltpu.VMEM((CH * 3,), jnp.int32),
        pltpu.VMEM((CH,), jnp.int32),
        pltpu.VMEM((CH * 3,), jnp.int32),
        pltpu.SemaphoreType.DMA,
        pltpu.SemaphoreType.DMA,
        pltpu.SemaphoreType.DMA,
        pltpu.SemaphoreType.DMA,
        pltpu.SemaphoreType.DMA,
        pltpu.SemaphoreType.DMA,
    ],
    compiler_params=_SC_PARAMS,
)(_asm_body)


def _mask_body(mb_ref, mask_ref):
    bits = mb_ref[...].reshape(1, N_TRI)
    shifts = jax.lax.broadcasted_iota(jnp.int32, (N_SENT, N_TRI), 0)
    mask_ref[...] = ((bits >> shifts) & 1) != 0


_mask_unpack = pl.pallas_call(
    _mask_body,
    out_shape=jax.ShapeDtypeStruct((N_SENT, N_TRI), jnp.bool_),
)


_EB = 13200


def _strip_body(enc_in, tid_in, enc_out, tid_out):
    enc_out[...] = enc_in[...]
    tid_out[...] = tid_in[...]


_strip = pl.pallas_call(
    _strip_body,
    grid=(N_TRI // _EB,),
    in_specs=[
        pl.BlockSpec((_EB, D), lambda i: (i, 0)),
        pl.BlockSpec((_EB, 3), lambda i: (i, 0)),
    ],
    out_specs=[
        pl.BlockSpec((_EB, D), lambda i: (i, 0)),
        pl.BlockSpec((_EB, 3), lambda i: (i, 0)),
    ],
    out_shape=[
        jax.ShapeDtypeStruct((N_TRI, D), jnp.float32),
        jax.ShapeDtypeStruct((N_TRI, 3), jnp.int32),
    ],
)


def kernel(concept_ids, edge_index, edge_relation, edge_weight,
           sent_concept_ids, concept_embedding, relation_embedding,
           self_loop_embedding, W1, b1, W2, b2, W_lin, b_lin):
    i32, f32 = jnp.int32, jnp.float32
    cid = concept_ids.astype(i32)
    src = edge_index[0].astype(i32)
    dst = edge_index[1].astype(i32)
    rel = edge_relation.astype(i32)
    w = edge_weight.astype(f32)

    npad_n = NODES_PAD - N_NODES
    vocab = concept_embedding.shape[0]
    cid_pad = jnp.concatenate([cid, vocab + jnp.arange(npad_n, dtype=i32)])
    ce_aug = jnp.concatenate(
        [concept_embedding.astype(f32), jnp.zeros((npad_n, D), f32)], axis=0)
    relv_flat = relation_embedding.astype(f32).reshape(-1)

    epad = NW * EPT_PAD - N_EDGES
    ar_e = jnp.arange(epad, dtype=i32)
    src_p = jnp.concatenate([src, N_NODES + (ar_e % npad_n)])
    dst_p = jnp.concatenate([dst, ar_e % N_NODES])
    rel_p = jnp.concatenate([rel, jnp.zeros((epad,), i32)])
    w_p = jnp.concatenate([w, jnp.zeros((epad,), f32)])
    x0h, aggf = _msgpass(ce_aug, cid_pad, src_p, dst_p, rel_p, w_p, relv_flat)

    relsl = jnp.concatenate(
        [relation_embedding.astype(f32), self_loop_embedding.astype(f32),
         jnp.zeros((1, D), f32)], axis=0)
    cid2d = cid_pad.reshape(NODES_PAD // D, D)
    A, C, R2, nm2d = _dense(
        x0h, aggf, W1.astype(f32), b1.astype(f32).reshape(1, D),
        W2.astype(f32), b2.astype(f32).reshape(1, D), W_lin.astype(f32),
        b_lin.astype(f32).reshape(1, D), relsl, cid2d,
        sent_concept_ids.astype(i32))

    nm_flat = nm2d.reshape(-1)
    ar_n = jnp.arange(N_NODES, dtype=i32)
    npad = TRI_PAD - N_TRI
    ar_p = jnp.arange(npad, dtype=i32) % N_NODES
    h_ext = jnp.concatenate([src, ar_n, ar_p])
    t_ext = jnp.concatenate([dst, ar_n, ar_p])
    r_ext = jnp.concatenate(
        [rel, jnp.full((N_NODES,), SELF_LOOP_ID, i32), jnp.zeros((npad,), i32)])
    w_ext = jnp.concatenate(
        [w, jnp.ones((N_NODES,), f32), jnp.zeros((npad,), f32)])

    enc_p, tid_flat, mb = _assemble(
        A, C, R2.reshape(-1), cid_pad, nm_flat, h_ext, t_ext, r_ext, w_ext)

    mask = _mask_unpack(mb[:N_TRI])
    enc, tid = _strip(enc_p, tid_flat.reshape(TRI_PAD, 3))
    return enc, mask, tid
